# baseline (speedup 1.0000x reference)
import functools

import jax
import jax.numpy as jnp
from jax import lax
from jax.experimental import pallas as pl
from jax.experimental.pallas import tpu as pltpu
from jax.experimental.pallas import tpu_sc as plsc

NC = 2
NS = 16
NW = NC * NS
L = 16

SUB = 128
NSUB = 5
CHUNK = SUB * NSUB


def _make_kernel(B, V, D, T):
    b_per_w = B // NW
    n_chunks = b_per_w // CHUNK
    assert b_per_w * NW == B and n_chunks * CHUNK == b_per_w and D % L == 0

    mesh = plsc.VectorSubcoreMesh(
        core_axis_name="c", subcore_axis_name="s", num_cores=NC, num_subcores=NS
    )

    @functools.partial(
        pl.kernel,
        mesh=mesh,
        out_type=jax.ShapeDtypeStruct((B, D), jnp.float32),
        scratch_types=[
            pltpu.VMEM((CHUNK,), jnp.int32),
            pltpu.VMEM((CHUNK, D), jnp.float32),
            pltpu.VMEM((T, D), jnp.float32),
            pltpu.SemaphoreType.DMA,
        ],
        compiler_params=pltpu.CompilerParams(use_tc_tiling_on_sc=False),
    )
    def k(idx_hbm, tok_hbm, pos_hbm, out_hbm, idx_v, buf_v, pos_v, sem):
        wid = lax.axis_index("s") * NC + lax.axis_index("c")
        base = wid * b_per_w
        pltpu.sync_copy(pos_hbm, pos_v)

        def chunk_body(c, carry):
            off = base + c * CHUNK
            pltpu.sync_copy(idx_hbm.at[pl.ds(off, CHUNK)], idx_v)
            copies = [
                pltpu.async_copy(
                    tok_hbm.at[idx_v.at[pl.ds(j * SUB, SUB)]],
                    buf_v.at[pl.ds(j * SUB, SUB)],
                    sem,
                )
                for j in range(NSUB)
            ]
            for cp in copies:
                cp.wait()

            def add_body(r, carry2):
                pr = lax.rem(c * CHUNK + r, T)
                for g in range(D // L):
                    v = pos_v[pr, pl.ds(g * L, L)]
                    plsc.addupdate(buf_v.at[r, pl.ds(g * L, L)], v)
                return carry2

            lax.fori_loop(0, CHUNK, add_body, 0, unroll=4)
            pltpu.sync_copy(buf_v, out_hbm.at[pl.ds(off, CHUNK)])
            return carry

        lax.fori_loop(0, n_chunks, chunk_body, 0)

    return k


def kernel(inputs, token_table, pos_table):
    Bt, T = inputs.shape
    V, D = token_table.shape
    B = Bt * T
    idx_flat = jnp.reshape(inputs.astype(jnp.int32), (B,))
    k = _make_kernel(B, V, D, T)
    out = k(idx_flat, token_table, pos_table)
    return jnp.reshape(out, (Bt, T, D))


# speedup vs baseline: 1.0435x; 1.0435x over previous
import functools

import jax
import jax.numpy as jnp
from jax import lax
from jax.experimental import pallas as pl
from jax.experimental.pallas import tpu as pltpu
from jax.experimental.pallas import tpu_sc as plsc

NC = 2
NS = 16
NW = NC * NS

SUB = 128
NSUB = 5
CHUNK = SUB * NSUB


def _sc_gather(B, V, D):
    b_per_w = B // NW
    n_chunks = b_per_w // CHUNK
    assert b_per_w * NW == B and n_chunks * CHUNK == b_per_w

    mesh = plsc.VectorSubcoreMesh(
        core_axis_name="c", subcore_axis_name="s", num_cores=NC, num_subcores=NS
    )

    @functools.partial(
        pl.kernel,
        mesh=mesh,
        out_type=jax.ShapeDtypeStruct((B, D), jnp.float32),
        scratch_types=[
            pltpu.VMEM((2, CHUNK), jnp.int32),
            pltpu.VMEM((2, CHUNK, D), jnp.float32),
            pltpu.SemaphoreType.DMA,
            pltpu.SemaphoreType.DMA,
            pltpu.SemaphoreType.DMA,
            pltpu.SemaphoreType.DMA,
        ],
        compiler_params=pltpu.CompilerParams(use_tc_tiling_on_sc=False),
    )
    def k(idx_hbm, tok_hbm, out_hbm, idx_v, buf_v, g0, g1, w0, w1):
        wid = lax.axis_index("s") * NC + lax.axis_index("c")
        base = wid * b_per_w
        gsem = [g0, g1]
        wsem = [w0, w1]
        gd = {}
        wd = {}

        def start(c):
            s = c % 2
            off = base + c * CHUNK
            pltpu.sync_copy(idx_hbm.at[pl.ds(off, CHUNK)], idx_v.at[s])
            gd[s] = [
                pltpu.async_copy(
                    tok_hbm.at[idx_v.at[s].at[pl.ds(j * SUB, SUB)]],
                    buf_v.at[s].at[pl.ds(j * SUB, SUB)],
                    gsem[s],
                )
                for j in range(NSUB)
            ]

        start(0)
        for c in range(n_chunks):
            s = c % 2
            if c + 1 < n_chunks:
                if c >= 1:
                    wd[(c + 1) % 2].wait()
                start(c + 1)
            for cp in gd[s]:
                cp.wait()
            wd[s] = pltpu.async_copy(
                buf_v.at[s], out_hbm.at[pl.ds(base + c * CHUNK, CHUNK)], wsem[s]
            )
        wd[(n_chunks - 2) % 2].wait()
        wd[(n_chunks - 1) % 2].wait()

    return k


def _tc_fixup(Bt, T, D):
    TB = 8
    BB = 256

    def body(g_ref, pos_ref, out_ref):
        x = g_ref[...]
        for t in range(TB):
            out_ref[t] = jnp.transpose(x[:, t, :], (1, 0)) + pos_ref[t][:, None]

    return pl.pallas_call(
        body,
        grid=(T // TB, Bt // BB),
        in_specs=[
            pl.BlockSpec((BB, TB, D), lambda ti, bi: (bi, ti, 0)),
            pl.BlockSpec((TB, D), lambda ti, bi: (ti, 0)),
        ],
        out_specs=pl.BlockSpec((TB, D, BB), lambda ti, bi: (ti, 0, bi)),
        out_shape=jax.ShapeDtypeStruct((T, D, Bt), jnp.float32),
    )


def kernel(inputs, token_table, pos_table):
    Bt, T = inputs.shape
    V, D = token_table.shape
    B = Bt * T
    idx_flat = jnp.reshape(inputs.astype(jnp.int32), (B,))
    gathered = _sc_gather(B, V, D)(idx_flat, token_table)
    g3 = jnp.reshape(gathered, (Bt, T, D))
    out_t = _tc_fixup(Bt, T, D)(g3, pos_table)
    return jnp.transpose(out_t, (2, 0, 1))
